# baseline (speedup 1.0000x reference)
import functools

import jax
import jax.numpy as jnp
from jax import lax
from jax.experimental import pallas as pl
from jax.experimental.pallas import tpu as pltpu

_BM = 200
_NSLOTS = 4


def _gcn_body(n, seq_ref, w_ref, adj_ref, b_ref, a_ref, out_ref,
              bufs, obufs, in_sems, out_sems):
    nchunks = n // _BM

    def in_copy(c):
        return pltpu.make_async_copy(
            adj_ref.at[pl.ds(c * _BM, _BM), :],
            bufs.at[c % _NSLOTS],
            in_sems.at[c % _NSLOTS],
        )

    def out_copy(c):
        return pltpu.make_async_copy(
            obufs.at[c % 2],
            out_ref.at[pl.ds(c * _BM, _BM), :],
            out_sems.at[c % 2],
        )

    for c in range(_NSLOTS - 1):
        in_copy(c).start()

    def step(c, _):
        in_copy(c).wait()
        agg = jnp.dot(bufs[c % _NSLOTS], seq_ref[...],
                      preferred_element_type=jnp.float32)

        @pl.when(c + _NSLOTS - 1 < nchunks)
        def _():
            in_copy(c + _NSLOTS - 1).start()

        acc = lax.dot_general(
            agg, w_ref[...], (((1,), (1,)), ((), ())),
            preferred_element_type=jnp.float32,
        )
        acc = acc + b_ref[...]

        @pl.when(c >= 2)
        def _():
            out_copy(c - 2).wait()

        obufs[c % 2] = jnp.where(acc >= 0, acc, a_ref[0] * acc)
        out_copy(c).start()
        return ()

    lax.fori_loop(0, nchunks, step, (), unroll=False)
    out_copy(nchunks - 2).wait()
    out_copy(nchunks - 1).wait()


def kernel(seq, adj, du, W, b, prelu_a):
    del du
    _, n, d_in = seq.shape
    d_out = W.shape[0]
    seq2 = seq.reshape(n, d_in)
    adj2 = adj.reshape(n, n)

    out = pl.pallas_call(
        functools.partial(_gcn_body, n),
        in_specs=[
            pl.BlockSpec((n, d_in), lambda: (0, 0)),
            pl.BlockSpec((d_out, d_in), lambda: (0, 0)),
            pl.BlockSpec(memory_space=pl.ANY),
            pl.BlockSpec((d_out,), lambda: (0,)),
            pl.BlockSpec((1,), lambda: (0,)),
        ],
        out_specs=pl.BlockSpec(memory_space=pl.ANY),
        out_shape=jax.ShapeDtypeStruct((n, d_out), jnp.float32),
        scratch_shapes=[
            pltpu.VMEM((_NSLOTS, _BM, n), jnp.float32),
            pltpu.VMEM((2, _BM, d_out), jnp.float32),
            pltpu.SemaphoreType.DMA((_NSLOTS,)),
            pltpu.SemaphoreType.DMA((2,)),
        ],
    )(seq2, W, adj2, b, prelu_a)
    return out.reshape(1, n, d_out)


# speedup vs baseline: 1.0096x; 1.0096x over previous
import jax
import jax.numpy as jnp
from jax import lax
from jax.experimental import pallas as pl

_BM = 400


def _gcn_body(seq_ref, w_ref, adj_ref, b_ref, a_ref, out_ref):
    agg = jnp.dot(
        adj_ref[...].astype(jnp.bfloat16),
        seq_ref[...].astype(jnp.bfloat16),
        preferred_element_type=jnp.float32,
    )
    acc = lax.dot_general(
        agg, w_ref[...], (((1,), (1,)), ((), ())),
        preferred_element_type=jnp.float32,
    )
    acc = acc + b_ref[...]
    out_ref[...] = jnp.where(acc >= 0, acc, a_ref[0] * acc)


def kernel(seq, adj, du, W, b, prelu_a):
    del du
    _, n, d_in = seq.shape
    d_out = W.shape[0]
    seq2 = seq.reshape(n, d_in)
    adj2 = adj.reshape(n, n)

    out = pl.pallas_call(
        _gcn_body,
        grid=(n // _BM,),
        in_specs=[
            pl.BlockSpec((n, d_in), lambda i: (0, 0)),
            pl.BlockSpec((d_out, d_in), lambda i: (0, 0)),
            pl.BlockSpec((_BM, n), lambda i: (i, 0)),
            pl.BlockSpec((d_out,), lambda i: (0,)),
            pl.BlockSpec((1,), lambda i: (0,)),
        ],
        out_specs=pl.BlockSpec((_BM, d_out), lambda i: (i, 0)),
        out_shape=jax.ShapeDtypeStruct((n, d_out), jnp.float32),
    )(seq2, W, adj2, b, prelu_a)
    return out.reshape(1, n, d_out)


# speedup vs baseline: 1.0133x; 1.0037x over previous
import jax
import jax.numpy as jnp
from jax import lax
from jax.experimental import pallas as pl
from jax.experimental.pallas import tpu as pltpu

_BM = 400


def _gcn_body(seq_ref, w_ref, adj_ref, b_ref, a_ref, out_ref):
    agg = jnp.dot(adj_ref[...], seq_ref[...], preferred_element_type=jnp.float32)
    acc = lax.dot_general(
        agg, w_ref[...], (((1,), (1,)), ((), ())),
        preferred_element_type=jnp.float32,
    )
    acc = acc + b_ref[...]
    out_ref[...] = jnp.where(acc >= 0, acc, a_ref[0] * acc)


def kernel(seq, adj, du, W, b, prelu_a):
    del du
    _, n, d_in = seq.shape
    d_out = W.shape[0]
    seq2 = seq.reshape(n, d_in)
    adj2 = adj.reshape(n, n)

    out = pl.pallas_call(
        _gcn_body,
        grid=(n // _BM,),
        in_specs=[
            pl.BlockSpec((n, d_in), lambda i: (0, 0)),
            pl.BlockSpec((d_out, d_in), lambda i: (0, 0)),
            pl.BlockSpec((_BM, n), lambda i: (i, 0)),
            pl.BlockSpec((d_out,), lambda i: (0,)),
            pl.BlockSpec((1,), lambda i: (0,)),
        ],
        out_specs=pl.BlockSpec((_BM, d_out), lambda i: (i, 0)),
        out_shape=jax.ShapeDtypeStruct((n, d_out), jnp.float32),
        compiler_params=pltpu.CompilerParams(
            dimension_semantics=("parallel",),
        ),
    )(seq2, W, adj2, b, prelu_a)
    return out.reshape(1, n, d_out)
